# Initial kernel scaffold; baseline (speedup 1.0000x reference)
import jax
import jax.numpy as jnp
from jax.experimental import pallas as pl


def kernel(x, edge_index, edges_prob, class_idx, W, a):
    raise NotImplementedError("write your pallas kernel here")


# speedup vs baseline: 6.1173x; 6.1173x over previous
import functools

import jax
import jax.numpy as jnp
from jax import lax
from jax.experimental import pallas as pl
from jax.experimental.pallas import tpu as pltpu
from jax.experimental.pallas import tpu_sc as plsc

NC = 2
NS = 16
L = 16
NW = NC * NS
GCH = 128
IMAX = 2**31 - 1


def _phase1_body(N, C, EPW, SP, CAP, n_comp, n_zero,
                 src_hbm, dst_hbm, prob_hbm, partp_hbm, parte_hbm, partd_hbm,
                 srcv, dstv, linv, elv, probv, bptr, lp, le, ld, sem):
    wid = lax.axis_index("s") * NC + lax.axis_index("c")
    base = wid * EPW
    it16 = lax.iota(jnp.int32, L)

    srcv[pl.ds(SP - L, L)] = jnp.full((L,), N, jnp.int32)
    pltpu.sync_copy(src_hbm.at[pl.ds(base, EPW)], srcv.at[pl.ds(0, EPW)])
    pltpu.sync_copy(dst_hbm.at[pl.ds(base, EPW)], dstv.at[pl.ds(0, EPW)])

    def zero_body(i, _):
        linv[pl.ds(i * L, L)] = jnp.zeros((L,), jnp.int32)
        return 0
    lax.fori_loop(0, n_zero, zero_body, 0)

    def comp_body(c, off):
        s = srcv[pl.ds(c * L, L)]
        d = dstv[pl.ds(c * L, L)]
        m = s < C
        lin = s * N + d
        e = base + c * L + it16
        plsc.store_compressed(linv.at[pl.ds(off, L)], lin, mask=m)
        plsc.store_compressed(elv.at[pl.ds(off, L)], e, mask=m)
        return off + jnp.max(plsc.all_reduce_population_count(m))
    nv = lax.fori_loop(0, n_comp, comp_body, jnp.int32(0))

    def g_body(g, _):
        pltpu.async_copy(prob_hbm.at[linv.at[pl.ds(g * GCH, GCH)]],
                         probv.at[pl.ds(g * GCH, GCH)], sem).wait()
        return 0
    lax.fori_loop(0, (nv + GCH - 1) // GCH, g_body, 0)

    def init_body(i, _):
        bptr[pl.ds(i * L, L)] = jnp.full((L,), -1, jnp.int32)
        return 0
    lax.fori_loop(0, C // L, init_body, 0)

    def k_body(k, _):
        j = k * L + it16
        lm = j < nv
        lin = linv[pl.ds(k * L, L)]
        e = elv[pl.ds(k * L, L)]
        p = probv[pl.ds(k * L, L)]
        c = lin // N

        def cond(act):
            return jnp.max(act) > 0

        def body(act):
            am = act > 0
            cur = plsc.load_gather(bptr, [c], mask=am)
            hasc = am & (cur >= 0)
            safe = jnp.maximum(cur, 0)
            curp = plsc.load_gather(probv, [safe], mask=hasc)
            cure = plsc.load_gather(elv, [safe], mask=hasc)
            better = (cur < 0) | (p > curp) | ((p == curp) & (e < cure))
            nact = am & better
            plsc.store_scatter(bptr, [c], j, mask=nact)
            return nact.astype(jnp.int32)

        lax.while_loop(cond, body, lm.astype(jnp.int32))
        return 0
    lax.fori_loop(0, (nv + L - 1) // L, k_body, 0)

    def f_body(i, _):
        ptr = bptr[pl.ds(i * L, L)]
        m = ptr >= 0
        safe = jnp.maximum(ptr, 0)
        pv = plsc.load_gather(probv, [safe], mask=m)
        ev = plsc.load_gather(elv, [safe], mask=m)
        lv = plsc.load_gather(linv, [safe], mask=m)
        lp[pl.ds(i * L, L)] = jnp.where(m, pv, jnp.float32(-1.0))
        le[pl.ds(i * L, L)] = jnp.where(m, ev, IMAX)
        ld[pl.ds(i * L, L)] = jnp.where(m, lv % N, 0)
        return 0
    lax.fori_loop(0, C // L, f_body, 0)

    pltpu.sync_copy(lp, partp_hbm.at[wid])
    pltpu.sync_copy(le, parte_hbm.at[wid])
    pltpu.sync_copy(ld, partd_hbm.at[wid])


def _phase2_body(C, partp_hbm, parte_hbm, partd_hbm, x_hbm,
                 xrows_hbm, hasnb_hbm, pp, pe, pd, idxv, hv, rows, sem):
    wid = lax.axis_index("s") * NC + lax.axis_index("c")

    @pl.when(wid < C // L)
    def _():
        pltpu.sync_copy(partp_hbm, pp)
        pltpu.sync_copy(parte_hbm, pe)
        pltpu.sync_copy(partd_hbm, pd)
        colbase = wid * L

        bp = jnp.full((L,), -2.0, jnp.float32)
        be = jnp.zeros((L,), jnp.int32)
        bd = jnp.zeros((L,), jnp.int32)
        for r in range(NW):
            pv = pp[r, pl.ds(colbase, L)]
            ev = pe[r, pl.ds(colbase, L)]
            dv = pd[r, pl.ds(colbase, L)]
            better = (pv > bp) | ((pv == bp) & (ev < be))
            bp = jnp.where(better, pv, bp)
            be = jnp.where(better, ev, be)
            bd = jnp.where(better, dv, bd)

        has = bp >= 0.0
        idxv[...] = jnp.where(has, bd, 0)
        hv[...] = has.astype(jnp.int32)
        pltpu.async_copy(x_hbm.at[idxv], rows, sem).wait()
        pltpu.sync_copy(rows, xrows_hbm.at[pl.ds(colbase, L)])
        pltpu.sync_copy(hv, hasnb_hbm.at[pl.ds(colbase, L)])


def _phase3_body(xr_ref, w_ref, hn_ref, fb_ref, o_ref):
    xw = jnp.dot(xr_ref[...], w_ref[...], preferred_element_type=jnp.float32)
    act = jnp.where(xw > 0.0, xw, jnp.exp(xw) - 1.0)
    o_ref[...] = jnp.where(hn_ref[...] > 0, act, fb_ref[...])


@functools.lru_cache(maxsize=None)
def _build(N, E, C, IN_DIM, OUT_DIM):
    assert E % NW == 0 and (E // NW) % 8 == 0 and C % L == 0
    EPW = E // NW
    n_comp = (EPW + L - 1) // L
    SP = n_comp * L
    CAP = ((SP + L) + GCH - 1) // GCH * GCH
    n_zero = CAP // L
    mesh = plsc.VectorSubcoreMesh(core_axis_name="c", subcore_axis_name="s",
                                  num_cores=NC, num_subcores=NS)

    p1 = pl.kernel(
        functools.partial(_phase1_body, N, C, EPW, SP, CAP, n_comp, n_zero),
        out_type=(jax.ShapeDtypeStruct((NW, C), jnp.float32),
                  jax.ShapeDtypeStruct((NW, C), jnp.int32),
                  jax.ShapeDtypeStruct((NW, C), jnp.int32)),
        mesh=mesh,
        scratch_types=[
            pltpu.VMEM((SP,), jnp.int32),
            pltpu.VMEM((SP,), jnp.int32),
            pltpu.VMEM((CAP,), jnp.int32),
            pltpu.VMEM((CAP,), jnp.int32),
            pltpu.VMEM((CAP,), jnp.float32),
            pltpu.VMEM((C,), jnp.int32),
            pltpu.VMEM((C,), jnp.float32),
            pltpu.VMEM((C,), jnp.int32),
            pltpu.VMEM((C,), jnp.int32),
            pltpu.SemaphoreType.DMA,
        ],
        compiler_params=pltpu.CompilerParams(needs_layout_passes=False),
    )

    p2 = pl.kernel(
        functools.partial(_phase2_body, C),
        out_type=(jax.ShapeDtypeStruct((C, IN_DIM), jnp.float32),
                  jax.ShapeDtypeStruct((C,), jnp.int32)),
        mesh=mesh,
        scratch_types=[
            pltpu.VMEM((NW, C), jnp.float32),
            pltpu.VMEM((NW, C), jnp.int32),
            pltpu.VMEM((NW, C), jnp.int32),
            pltpu.VMEM((L,), jnp.int32),
            pltpu.VMEM((L,), jnp.int32),
            pltpu.VMEM((L, IN_DIM), jnp.float32),
            pltpu.SemaphoreType.DMA,
        ],
        compiler_params=pltpu.CompilerParams(needs_layout_passes=False),
    )

    p3 = pl.pallas_call(
        _phase3_body,
        out_shape=jax.ShapeDtypeStruct((C, OUT_DIM), jnp.float32),
    )
    return p1, p2, p3


def kernel(x, edge_index, edges_prob, class_idx, W, a):
    N, IN_DIM = x.shape
    E = edge_index.shape[1]
    C = class_idx.shape[0]
    OUT_DIM = W.shape[1]
    p1, p2, p3 = _build(N, E, C, IN_DIM, OUT_DIM)

    src = edge_index[0]
    dst = edge_index[1]
    prob_flat = edges_prob.reshape(-1)

    partp, parte, partd = p1(src, dst, prob_flat)
    xrows, hasnb = p2(partp, parte, partd, x)

    rk = jax.random.key(1)
    h_rand = jax.vmap(
        lambda i: jax.random.normal(jax.random.fold_in(rk, i), (OUT_DIM,),
                                    dtype=jnp.float32))(class_idx)
    fallback = jax.nn.elu(h_rand)

    return p3(xrows, W, hasnb.reshape(C, 1), fallback)


# speedup vs baseline: 34.5807x; 5.6529x over previous
import functools

import jax
import jax.numpy as jnp
from jax import lax
from jax.experimental import pallas as pl
from jax.experimental.pallas import tpu as pltpu
from jax.experimental.pallas import tpu_sc as plsc

NC = 2
NS = 16
L = 16
NW = NC * NS
GCH = 128
IMAX = 2**31 - 1


def _phase1_body(N, C, EPW, SP, CAP, n_comp, n_zero,
                 src_hbm, dst_hbm, prob_hbm, partp_hbm, parte_hbm, partd_hbm,
                 srcv, dstv, linv, elv, probv, bptr, lp, le, ld, sem):
    wid = lax.axis_index("s") * NC + lax.axis_index("c")
    base = wid * EPW
    it16 = lax.iota(jnp.int32, L)

    srcv[pl.ds(SP - L, L)] = jnp.full((L,), N, jnp.int32)
    pltpu.sync_copy(src_hbm.at[pl.ds(base, EPW)], srcv.at[pl.ds(0, EPW)])
    pltpu.sync_copy(dst_hbm.at[pl.ds(base, EPW)], dstv.at[pl.ds(0, EPW)])

    def zero_body(i, _):
        linv[pl.ds(i * L, L)] = jnp.zeros((L,), jnp.int32)
        return 0
    lax.fori_loop(0, n_zero, zero_body, 0)

    def comp_body(c, off):
        s = srcv[pl.ds(c * L, L)]
        d = dstv[pl.ds(c * L, L)]
        m = s < C
        lin = s * N + d
        e = base + c * L + it16
        plsc.store_compressed(linv.at[pl.ds(off, L)], lin, mask=m)
        plsc.store_compressed(elv.at[pl.ds(off, L)], e, mask=m)
        return off + jnp.max(plsc.all_reduce_population_count(m))
    nv = lax.fori_loop(0, n_comp, comp_body, jnp.int32(0))

    def g_body(g, _):
        pltpu.async_copy(prob_hbm.at[linv.at[pl.ds(g * GCH, GCH)]],
                         probv.at[pl.ds(g * GCH, GCH)], sem).wait()
        return 0
    lax.fori_loop(0, (nv + GCH - 1) // GCH, g_body, 0)

    def init_body(i, _):
        bptr[pl.ds(i * L, L)] = jnp.full((L,), -1, jnp.int32)
        return 0
    lax.fori_loop(0, C // L, init_body, 0)

    def k_body(k, _):
        j = k * L + it16
        lm = j < nv
        lin = linv[pl.ds(k * L, L)]
        e = elv[pl.ds(k * L, L)]
        p = probv[pl.ds(k * L, L)]
        c = lin // N

        def cond(act):
            return jnp.max(act) > 0

        def body(act):
            am = act > 0
            cur = plsc.load_gather(bptr, [c], mask=am)
            hasc = am & (cur >= 0)
            safe = jnp.maximum(cur, 0)
            curp = plsc.load_gather(probv, [safe], mask=hasc)
            cure = plsc.load_gather(elv, [safe], mask=hasc)
            better = (cur < 0) | (p > curp) | ((p == curp) & (e < cure))
            nact = am & better
            plsc.store_scatter(bptr, [c], j, mask=nact)
            return nact.astype(jnp.int32)

        lax.while_loop(cond, body, lm.astype(jnp.int32))
        return 0
    lax.fori_loop(0, (nv + L - 1) // L, k_body, 0)

    def f_body(i, _):
        ptr = bptr[pl.ds(i * L, L)]
        m = ptr >= 0
        safe = jnp.maximum(ptr, 0)
        pv = plsc.load_gather(probv, [safe], mask=m)
        ev = plsc.load_gather(elv, [safe], mask=m)
        lv = plsc.load_gather(linv, [safe], mask=m)
        lp[pl.ds(i * L, L)] = jnp.where(m, pv, jnp.float32(-1.0))
        le[pl.ds(i * L, L)] = jnp.where(m, ev, IMAX)
        ld[pl.ds(i * L, L)] = jnp.where(m, lv % N, 0)
        return 0
    lax.fori_loop(0, C // L, f_body, 0)

    pltpu.sync_copy(lp, partp_hbm.at[wid])
    pltpu.sync_copy(le, parte_hbm.at[wid])
    pltpu.sync_copy(ld, partd_hbm.at[wid])


def _phase2_body(C, partp_hbm, parte_hbm, partd_hbm, x_hbm,
                 xrows_hbm, hasnb_hbm, pp, pe, pd, idxv, hv, rows, sem):
    wid = lax.axis_index("s") * NC + lax.axis_index("c")

    @pl.when(wid < C // L)
    def _():
        pltpu.sync_copy(partp_hbm, pp)
        pltpu.sync_copy(parte_hbm, pe)
        pltpu.sync_copy(partd_hbm, pd)
        colbase = wid * L

        bp = jnp.full((L,), -2.0, jnp.float32)
        be = jnp.zeros((L,), jnp.int32)
        bd = jnp.zeros((L,), jnp.int32)
        for r in range(NW):
            pv = pp[r, pl.ds(colbase, L)]
            ev = pe[r, pl.ds(colbase, L)]
            dv = pd[r, pl.ds(colbase, L)]
            better = (pv > bp) | ((pv == bp) & (ev < be))
            bp = jnp.where(better, pv, bp)
            be = jnp.where(better, ev, be)
            bd = jnp.where(better, dv, bd)

        has = bp >= 0.0
        idxv[...] = jnp.where(has, bd, 0)
        hv[...] = has.astype(jnp.int32)
        pltpu.async_copy(x_hbm.at[idxv], rows, sem).wait()
        pltpu.sync_copy(rows, xrows_hbm.at[pl.ds(colbase, L)])
        pltpu.sync_copy(hv, hasnb_hbm.at[pl.ds(colbase, L)])


def _phase3_body(xr_ref, w_ref, hn_ref, fb_ref, o_ref):
    xw = jnp.dot(xr_ref[...], w_ref[...], preferred_element_type=jnp.float32)
    act = jnp.where(xw > 0.0, xw, jnp.exp(xw) - 1.0)
    o_ref[...] = jnp.where(hn_ref[...] > 0, act, fb_ref[...])


@functools.lru_cache(maxsize=None)
def _build(N, E, C, IN_DIM, OUT_DIM):
    assert E % NW == 0 and (E // NW) % 8 == 0 and C % L == 0
    EPW = E // NW
    n_comp = (EPW + L - 1) // L
    SP = n_comp * L
    CAP = ((SP + L) + GCH - 1) // GCH * GCH
    n_zero = CAP // L
    mesh = plsc.VectorSubcoreMesh(core_axis_name="c", subcore_axis_name="s",
                                  num_cores=NC, num_subcores=NS)

    p1 = pl.kernel(
        functools.partial(_phase1_body, N, C, EPW, SP, CAP, n_comp, n_zero),
        out_type=(jax.ShapeDtypeStruct((NW, C), jnp.float32),
                  jax.ShapeDtypeStruct((NW, C), jnp.int32),
                  jax.ShapeDtypeStruct((NW, C), jnp.int32)),
        mesh=mesh,
        scratch_types=[
            pltpu.VMEM((SP,), jnp.int32),
            pltpu.VMEM((SP,), jnp.int32),
            pltpu.VMEM((CAP,), jnp.int32),
            pltpu.VMEM((CAP,), jnp.int32),
            pltpu.VMEM((CAP,), jnp.float32),
            pltpu.VMEM((C,), jnp.int32),
            pltpu.VMEM((C,), jnp.float32),
            pltpu.VMEM((C,), jnp.int32),
            pltpu.VMEM((C,), jnp.int32),
            pltpu.SemaphoreType.DMA,
        ],
        compiler_params=pltpu.CompilerParams(needs_layout_passes=False),
    )

    p2 = pl.kernel(
        functools.partial(_phase2_body, C),
        out_type=(jax.ShapeDtypeStruct((C, IN_DIM), jnp.float32),
                  jax.ShapeDtypeStruct((C,), jnp.int32)),
        mesh=mesh,
        scratch_types=[
            pltpu.VMEM((NW, C), jnp.float32),
            pltpu.VMEM((NW, C), jnp.int32),
            pltpu.VMEM((NW, C), jnp.int32),
            pltpu.VMEM((L,), jnp.int32),
            pltpu.VMEM((L,), jnp.int32),
            pltpu.VMEM((L, IN_DIM), jnp.float32),
            pltpu.SemaphoreType.DMA,
        ],
        compiler_params=pltpu.CompilerParams(needs_layout_passes=False),
    )

    p3 = pl.pallas_call(
        _phase3_body,
        out_shape=jax.ShapeDtypeStruct((C, OUT_DIM), jnp.float32),
    )
    return p1, p2, p3


def kernel(x, edge_index, edges_prob, class_idx, W, a):
    N, IN_DIM = x.shape
    E = edge_index.shape[1]
    C = class_idx.shape[0]
    OUT_DIM = W.shape[1]
    p1, p2, p3 = _build(N, E, C, IN_DIM, OUT_DIM)

    src = edge_index[0]
    dst = edge_index[1]
    prob_flat = edges_prob[:C].reshape(-1)

    partp, parte, partd = p1(src, dst, prob_flat)
    xrows, hasnb = p2(partp, parte, partd, x)

    rk = jax.random.key(1)
    h_rand = jax.vmap(
        lambda i: jax.random.normal(jax.random.fold_in(rk, i), (OUT_DIM,),
                                    dtype=jnp.float32))(class_idx)
    fallback = jax.nn.elu(h_rand)

    return p3(xrows, W, hasnb.reshape(C, 1), fallback)


# speedup vs baseline: 36.2020x; 1.0469x over previous
import functools

import jax
import jax.numpy as jnp
from jax import lax
from jax.experimental import pallas as pl
from jax.experimental.pallas import tpu as pltpu
from jax.experimental.pallas import tpu_sc as plsc

NC = 2
NS = 16
L = 16
NW = NC * NS
GCH = 128
IMAX = 2**31 - 1


def _phase1_body(N, C, EPW, SP, CAP, n_comp, n_zero,
                 src_hbm, dst_hbm, prob_hbm, partp_hbm, parte_hbm, partd_hbm,
                 srcv, dstv, linb, elv, probv, bptr, lp, le, ld, sem):
    wid = lax.axis_index("s") * NC + lax.axis_index("c")
    base = wid * EPW
    it16 = lax.iota(jnp.int32, L)

    srcv[pl.ds(SP - L, L)] = jnp.full((L,), N, jnp.int32)
    cp_s = pltpu.async_copy(src_hbm.at[pl.ds(base, EPW)],
                            srcv.at[pl.ds(0, EPW)], sem)
    cp_d = pltpu.async_copy(dst_hbm.at[pl.ds(base, EPW)],
                            dstv.at[pl.ds(0, EPW)], sem)
    cp_s.wait()
    cp_d.wait()

    def comp_body(c, off):
        s = srcv[pl.ds(c * L, L)]
        m = s < C

        def do_store():
            e = base + c * L + it16
            plsc.store_compressed(elv.at[pl.ds(off, L)], e, mask=m)
            return off + plsc.all_reduce_population_count(m)[0]

        return lax.cond(jnp.any(m), do_store, lambda: off)
    nv = lax.fori_loop(0, n_comp, comp_body, jnp.int32(0))

    def g_body(g, _):
        for i in range(GCH // L):
            e16 = elv[pl.ds(g * GCH + i * L, L)]
            eloc = jnp.clip(e16 - base, 0, SP - 1)
            s = plsc.load_gather(srcv, [eloc])
            d = plsc.load_gather(dstv, [eloc])
            linb[pl.ds(i * L, L)] = jnp.minimum(s, C - 1) * N + d
        pltpu.async_copy(prob_hbm.at[linb],
                         probv.at[pl.ds(g * GCH, GCH)], sem).wait()
        return 0
    lax.fori_loop(0, (nv + GCH - 1) // GCH, g_body, 0)

    def init_body(i, _):
        bptr[pl.ds(i * L, L)] = jnp.full((L,), -1, jnp.int32)
        return 0
    lax.fori_loop(0, C // L, init_body, 0)

    def k_body(k, _):
        j = k * L + it16
        lm = j < nv
        e = elv[pl.ds(k * L, L)]
        p = probv[pl.ds(k * L, L)]
        eloc = jnp.clip(e - base, 0, SP - 1)
        c = jnp.minimum(plsc.load_gather(srcv, [eloc]), C - 1)

        def cond(act):
            return jnp.max(act) > 0

        def body(act):
            am = act > 0
            cur = plsc.load_gather(bptr, [c], mask=am)
            hasc = am & (cur >= 0)
            safe = jnp.maximum(cur, 0)
            curp = plsc.load_gather(probv, [safe], mask=hasc)
            cure = plsc.load_gather(elv, [safe], mask=hasc)
            better = (cur < 0) | (p > curp) | ((p == curp) & (e < cure))
            nact = am & better
            plsc.store_scatter(bptr, [c], j, mask=nact)
            return nact.astype(jnp.int32)

        lax.while_loop(cond, body, lm.astype(jnp.int32))
        return 0
    lax.fori_loop(0, (nv + L - 1) // L, k_body, 0)

    def f_body(i, _):
        ptr = bptr[pl.ds(i * L, L)]
        m = ptr >= 0
        safe = jnp.maximum(ptr, 0)
        pv = plsc.load_gather(probv, [safe], mask=m)
        ev = plsc.load_gather(elv, [safe], mask=m)
        eloc = jnp.clip(ev - base, 0, SP - 1)
        dv = plsc.load_gather(dstv, [eloc], mask=m)
        lp[pl.ds(i * L, L)] = jnp.where(m, pv, jnp.float32(-1.0))
        le[pl.ds(i * L, L)] = jnp.where(m, ev, IMAX)
        ld[pl.ds(i * L, L)] = jnp.where(m, dv, 0)
        return 0
    lax.fori_loop(0, C // L, f_body, 0)

    pltpu.sync_copy(lp, partp_hbm.at[wid])
    pltpu.sync_copy(le, parte_hbm.at[wid])
    pltpu.sync_copy(ld, partd_hbm.at[wid])


def _phase2_body(C, partp_hbm, parte_hbm, partd_hbm, x_hbm,
                 xrows_hbm, hasnb_hbm, pp, pe, pd, idxv, hv, rows, sem):
    wid = lax.axis_index("s") * NC + lax.axis_index("c")

    @pl.when(wid < C // L)
    def _():
        pltpu.sync_copy(partp_hbm, pp)
        pltpu.sync_copy(parte_hbm, pe)
        pltpu.sync_copy(partd_hbm, pd)
        colbase = wid * L

        bp = jnp.full((L,), -2.0, jnp.float32)
        be = jnp.zeros((L,), jnp.int32)
        bd = jnp.zeros((L,), jnp.int32)
        for r in range(NW):
            pv = pp[r, pl.ds(colbase, L)]
            ev = pe[r, pl.ds(colbase, L)]
            dv = pd[r, pl.ds(colbase, L)]
            better = (pv > bp) | ((pv == bp) & (ev < be))
            bp = jnp.where(better, pv, bp)
            be = jnp.where(better, ev, be)
            bd = jnp.where(better, dv, bd)

        has = bp >= 0.0
        idxv[...] = jnp.where(has, bd, 0)
        hv[...] = has.astype(jnp.int32)
        pltpu.async_copy(x_hbm.at[idxv], rows, sem).wait()
        pltpu.sync_copy(rows, xrows_hbm.at[pl.ds(colbase, L)])
        pltpu.sync_copy(hv, hasnb_hbm.at[pl.ds(colbase, L)])


def _phase3_body(xr_ref, w_ref, hn_ref, fb_ref, o_ref):
    xw = jnp.dot(xr_ref[...], w_ref[...], preferred_element_type=jnp.float32)
    act = jnp.where(xw > 0.0, xw, jnp.exp(xw) - 1.0)
    o_ref[...] = jnp.where(hn_ref[...] > 0, act, fb_ref[...])


@functools.lru_cache(maxsize=None)
def _build(N, E, C, IN_DIM, OUT_DIM):
    assert E % NW == 0 and (E // NW) % 8 == 0 and C % L == 0
    EPW = E // NW
    n_comp = (EPW + L - 1) // L
    SP = n_comp * L
    CAP = ((SP + L) + GCH - 1) // GCH * GCH
    n_zero = CAP // L
    mesh = plsc.VectorSubcoreMesh(core_axis_name="c", subcore_axis_name="s",
                                  num_cores=NC, num_subcores=NS)

    p1 = pl.kernel(
        functools.partial(_phase1_body, N, C, EPW, SP, CAP, n_comp, n_zero),
        out_type=(jax.ShapeDtypeStruct((NW, C), jnp.float32),
                  jax.ShapeDtypeStruct((NW, C), jnp.int32),
                  jax.ShapeDtypeStruct((NW, C), jnp.int32)),
        mesh=mesh,
        scratch_types=[
            pltpu.VMEM((SP,), jnp.int32),
            pltpu.VMEM((SP,), jnp.int32),
            pltpu.VMEM((GCH,), jnp.int32),
            pltpu.VMEM((CAP,), jnp.int32),
            pltpu.VMEM((CAP,), jnp.float32),
            pltpu.VMEM((C,), jnp.int32),
            pltpu.VMEM((C,), jnp.float32),
            pltpu.VMEM((C,), jnp.int32),
            pltpu.VMEM((C,), jnp.int32),
            pltpu.SemaphoreType.DMA,
        ],
        compiler_params=pltpu.CompilerParams(needs_layout_passes=False),
    )

    p2 = pl.kernel(
        functools.partial(_phase2_body, C),
        out_type=(jax.ShapeDtypeStruct((C, IN_DIM), jnp.float32),
                  jax.ShapeDtypeStruct((C,), jnp.int32)),
        mesh=mesh,
        scratch_types=[
            pltpu.VMEM((NW, C), jnp.float32),
            pltpu.VMEM((NW, C), jnp.int32),
            pltpu.VMEM((NW, C), jnp.int32),
            pltpu.VMEM((L,), jnp.int32),
            pltpu.VMEM((L,), jnp.int32),
            pltpu.VMEM((L, IN_DIM), jnp.float32),
            pltpu.SemaphoreType.DMA,
        ],
        compiler_params=pltpu.CompilerParams(needs_layout_passes=False),
    )

    p3 = pl.pallas_call(
        _phase3_body,
        out_shape=jax.ShapeDtypeStruct((C, OUT_DIM), jnp.float32),
    )
    return p1, p2, p3


def kernel(x, edge_index, edges_prob, class_idx, W, a):
    N, IN_DIM = x.shape
    E = edge_index.shape[1]
    C = class_idx.shape[0]
    OUT_DIM = W.shape[1]
    p1, p2, p3 = _build(N, E, C, IN_DIM, OUT_DIM)

    src = edge_index[0]
    dst = edge_index[1]
    prob_flat = edges_prob[:C].reshape(-1)

    partp, parte, partd = p1(src, dst, prob_flat)
    xrows, hasnb = p2(partp, parte, partd, x)

    rk = jax.random.key(1)
    h_rand = jax.vmap(
        lambda i: jax.random.normal(jax.random.fold_in(rk, i), (OUT_DIM,),
                                    dtype=jnp.float32))(class_idx)
    fallback = jax.nn.elu(h_rand)

    return p3(xrows, W, hasnb.reshape(C, 1), fallback)


# speedup vs baseline: 42.4676x; 1.1731x over previous
import functools

import jax
import jax.numpy as jnp
from jax import lax
from jax.experimental import pallas as pl
from jax.experimental.pallas import tpu as pltpu
from jax.experimental.pallas import tpu_sc as plsc

NC = 2
NS = 16
L = 16
NW = NC * NS
GCH = 128
IMAX = 2**31 - 1


def _phase1_body(N, C, EPW, SP, CAP, n_comp, n_zero,
                 ei_hbm, prob_hbm, partp_hbm, parte_hbm, partd_hbm,
    eiv, linb, elv, probv, bptr, lp, le, ld,
                 sem):
    wid = lax.axis_index("s") * NC + lax.axis_index("c")
    E = ei_hbm.shape[1]
    base = jnp.where(wid == NW - 1, E - SP, wid * EPW)
    it16 = lax.iota(jnp.int32, L)
    z16 = jnp.zeros((L,), jnp.int32)
    o16 = jnp.full((L,), 1, jnp.int32)

    pltpu.sync_copy(ei_hbm.at[:, pl.ds(base, SP)], eiv)

    def comp_body(c, off):
        s = eiv[0, pl.ds(c * L, L)]
        m = s < C

        def do_store():
            e = base + c * L + it16
            plsc.store_compressed(elv.at[pl.ds(off, L)], e, mask=m)
            return off + plsc.all_reduce_population_count(m)[0]

        return lax.cond(jnp.any(m), do_store, lambda: off)
    nv = lax.fori_loop(0, n_comp, comp_body, jnp.int32(0))

    def g_body(g, _):
        for i in range(GCH // L):
            e16 = elv[pl.ds(g * GCH + i * L, L)]
            eloc = jnp.clip(e16 - base, 0, SP - 1)
            s = plsc.load_gather(eiv, [z16, eloc])
            d = plsc.load_gather(eiv, [o16, eloc])
            linb[pl.ds(i * L, L)] = jnp.minimum(s, C - 1) * N + d
        pltpu.async_copy(prob_hbm.at[linb],
                         probv.at[pl.ds(g * GCH, GCH)], sem).wait()
        return 0
    lax.fori_loop(0, (nv + GCH - 1) // GCH, g_body, 0)

    def init_body(i, _):
        bptr[pl.ds(i * L, L)] = jnp.full((L,), -1, jnp.int32)
        return 0
    lax.fori_loop(0, C // L, init_body, 0)

    def k_body(k, _):
        j = k * L + it16
        lm = j < nv
        e = elv[pl.ds(k * L, L)]
        p = probv[pl.ds(k * L, L)]
        eloc = jnp.clip(e - base, 0, SP - 1)
        c = jnp.minimum(plsc.load_gather(eiv, [z16, eloc]), C - 1)

        def cond(act):
            return jnp.max(act) > 0

        def body(act):
            am = act > 0
            cur = plsc.load_gather(bptr, [c], mask=am)
            hasc = am & (cur >= 0)
            safe = jnp.maximum(cur, 0)
            curp = plsc.load_gather(probv, [safe], mask=hasc)
            cure = plsc.load_gather(elv, [safe], mask=hasc)
            better = (cur < 0) | (p > curp) | ((p == curp) & (e < cure))
            nact = am & better
            plsc.store_scatter(bptr, [c], j, mask=nact)
            return nact.astype(jnp.int32)

        lax.while_loop(cond, body, lm.astype(jnp.int32))
        return 0
    lax.fori_loop(0, (nv + L - 1) // L, k_body, 0)

    def f_body(i, _):
        ptr = bptr[pl.ds(i * L, L)]
        m = ptr >= 0
        safe = jnp.maximum(ptr, 0)
        pv = plsc.load_gather(probv, [safe], mask=m)
        ev = plsc.load_gather(elv, [safe], mask=m)
        eloc = jnp.clip(ev - base, 0, SP - 1)
        dv = plsc.load_gather(eiv, [o16, eloc], mask=m)
        lp[pl.ds(i * L, L)] = jnp.where(m, pv, jnp.float32(-1.0))
        le[pl.ds(i * L, L)] = jnp.where(m, ev, IMAX)
        ld[pl.ds(i * L, L)] = jnp.where(m, dv, 0)
        return 0
    lax.fori_loop(0, C // L, f_body, 0)

    pltpu.sync_copy(lp, partp_hbm.at[wid])
    pltpu.sync_copy(le, parte_hbm.at[wid])
    pltpu.sync_copy(ld, partd_hbm.at[wid])


def _phase2_body(C, partp_hbm, parte_hbm, partd_hbm, x_hbm,
                 xrows_hbm, hasnb_hbm, pp, pe, pd, idxv, hv, rows, sem):
    wid = lax.axis_index("s") * NC + lax.axis_index("c")

    @pl.when(wid < C // L)
    def _():
        pltpu.sync_copy(partp_hbm, pp)
        pltpu.sync_copy(parte_hbm, pe)
        pltpu.sync_copy(partd_hbm, pd)
        colbase = wid * L

        bp = jnp.full((L,), -2.0, jnp.float32)
        be = jnp.zeros((L,), jnp.int32)
        bd = jnp.zeros((L,), jnp.int32)
        for r in range(NW):
            pv = pp[r, pl.ds(colbase, L)]
            ev = pe[r, pl.ds(colbase, L)]
            dv = pd[r, pl.ds(colbase, L)]
            better = (pv > bp) | ((pv == bp) & (ev < be))
            bp = jnp.where(better, pv, bp)
            be = jnp.where(better, ev, be)
            bd = jnp.where(better, dv, bd)

        has = bp >= 0.0
        idxv[...] = jnp.where(has, bd, 0)
        hv[...] = has.astype(jnp.int32)
        pltpu.async_copy(x_hbm.at[idxv], rows, sem).wait()
        pltpu.sync_copy(rows, xrows_hbm.at[pl.ds(colbase, L)])
        pltpu.sync_copy(hv, hasnb_hbm.at[pl.ds(colbase, L)])


def _phase3_body(xr_ref, w_ref, hn_ref, fb_ref, o_ref):
    xw = jnp.dot(xr_ref[...], w_ref[...], preferred_element_type=jnp.float32)
    act = jnp.where(xw > 0.0, xw, jnp.exp(xw) - 1.0)
    o_ref[...] = jnp.where(hn_ref[...] > 0, act, fb_ref[...])


@functools.lru_cache(maxsize=None)
def _build(N, E, C, IN_DIM, OUT_DIM):
    assert C % L == 0
    SP = -(-(-(-E // NW)) // 128) * 128
    EPW = (E - SP) // (NW - 1) // 128 * 128
    assert EPW * (NW - 2) + SP >= E - SP and SP <= E
    n_comp = SP // L
    CAP = SP
    n_zero = 0
    mesh = plsc.VectorSubcoreMesh(core_axis_name="c", subcore_axis_name="s",
                                  num_cores=NC, num_subcores=NS)

    p1 = pl.kernel(
        functools.partial(_phase1_body, N, C, EPW, SP, CAP, n_comp, n_zero),
        out_type=(jax.ShapeDtypeStruct((NW, C), jnp.float32),
                  jax.ShapeDtypeStruct((NW, C), jnp.int32),
                  jax.ShapeDtypeStruct((NW, C), jnp.int32)),
        mesh=mesh,
        scratch_types=[
            pltpu.VMEM((2, SP), jnp.int32),
            pltpu.VMEM((GCH,), jnp.int32),
            pltpu.VMEM((CAP,), jnp.int32),
            pltpu.VMEM((CAP,), jnp.float32),
            pltpu.VMEM((C,), jnp.int32),
            pltpu.VMEM((C,), jnp.float32),
            pltpu.VMEM((C,), jnp.int32),
            pltpu.VMEM((C,), jnp.int32),
            pltpu.SemaphoreType.DMA,
        ],
        compiler_params=pltpu.CompilerParams(needs_layout_passes=False),
    )

    p2 = pl.kernel(
        functools.partial(_phase2_body, C),
        out_type=(jax.ShapeDtypeStruct((C, IN_DIM), jnp.float32),
                  jax.ShapeDtypeStruct((C,), jnp.int32)),
        mesh=mesh,
        scratch_types=[
            pltpu.VMEM((NW, C), jnp.float32),
            pltpu.VMEM((NW, C), jnp.int32),
            pltpu.VMEM((NW, C), jnp.int32),
            pltpu.VMEM((L,), jnp.int32),
            pltpu.VMEM((L,), jnp.int32),
            pltpu.VMEM((L, IN_DIM), jnp.float32),
            pltpu.SemaphoreType.DMA,
        ],
        compiler_params=pltpu.CompilerParams(needs_layout_passes=False),
    )

    p3 = pl.pallas_call(
        _phase3_body,
        out_shape=jax.ShapeDtypeStruct((C, OUT_DIM), jnp.float32),
    )
    return p1, p2, p3


def kernel(x, edge_index, edges_prob, class_idx, W, a):
    N, IN_DIM = x.shape
    E = edge_index.shape[1]
    C = class_idx.shape[0]
    OUT_DIM = W.shape[1]
    p1, p2, p3 = _build(N, E, C, IN_DIM, OUT_DIM)

    prob_flat = edges_prob[:C].reshape(-1)
    partp, parte, partd = p1(edge_index, prob_flat)
    xrows, hasnb = p2(partp, parte, partd, x)

    rk = jax.random.key(1)
    h_rand = jax.vmap(
        lambda i: jax.random.normal(jax.random.fold_in(rk, i), (OUT_DIM,),
                                    dtype=jnp.float32))(class_idx)
    fallback = jax.nn.elu(h_rand)

    return p3(xrows, W, hasnb.reshape(C, 1), fallback)
